# baseline (speedup 1.0000x reference)
import jax
import jax.numpy as jnp
from jax.experimental import pallas as pl
from jax.experimental.pallas import tpu as pltpu

IN_DIM = 1024
OUT_DIM = 1024
K = 2048
B_TOK = 8192
TM = 1024
CHUNK = 256
N_ACHUNK = K // CHUNK
N_BCHUNK = K // CHUNK


def _fused_body(x_ref, A_ref, B_ref, out_ref, inner_ref,
                An_s, Bb_s, atmp, btmp, asem, bsem):
    i = pl.program_id(0)

    @pl.when(i == 0)
    def _prep():
        def load_a(c, _):
            cp = pltpu.make_async_copy(
                A_ref.at[:, pl.ds(c * CHUNK, CHUNK)], atmp, asem)
            cp.start()
            cp.wait()
            a = atmp[...]
            inv = jax.lax.rsqrt(jnp.sum(a * a, axis=0, keepdims=True))
            An_s[:, pl.ds(c * CHUNK, CHUNK)] = (a * inv).astype(jnp.bfloat16)
            return 0

        jax.lax.fori_loop(0, N_ACHUNK, load_a, 0, unroll=False)

        def load_b(c, _):
            cp = pltpu.make_async_copy(
                B_ref.at[pl.ds(c * CHUNK, CHUNK), :], btmp, bsem)
            cp.start()
            cp.wait()
            Bb_s[pl.ds(c * CHUNK, CHUNK), :] = btmp[...].astype(jnp.bfloat16)
            return 0

        jax.lax.fori_loop(0, N_BCHUNK, load_b, 0, unroll=False)

    inner = jnp.dot(x_ref[...].astype(jnp.bfloat16), An_s[...],
                    preferred_element_type=jnp.float32)
    inner_ref[...] = inner
    out_ref[...] = jnp.dot(inner.astype(jnp.bfloat16), Bb_s[...],
                           preferred_element_type=jnp.float32)


def kernel(x, A, Bm):
    n_tiles = B_TOK // TM
    out, inner = pl.pallas_call(
        _fused_body,
        grid=(n_tiles,),
        in_specs=[
            pl.BlockSpec((TM, IN_DIM), lambda i: (i, 0)),
            pl.BlockSpec(memory_space=pl.ANY),
            pl.BlockSpec(memory_space=pl.ANY),
        ],
        out_specs=[
            pl.BlockSpec((TM, OUT_DIM), lambda i: (i, 0)),
            pl.BlockSpec((TM, K), lambda i: (i, 0)),
        ],
        out_shape=[
            jax.ShapeDtypeStruct((B_TOK, OUT_DIM), jnp.float32),
            jax.ShapeDtypeStruct((B_TOK, K), jnp.float32),
        ],
        scratch_shapes=[
            pltpu.VMEM((IN_DIM, K), jnp.bfloat16),
            pltpu.VMEM((K, OUT_DIM), jnp.bfloat16),
            pltpu.VMEM((IN_DIM, CHUNK), jnp.float32),
            pltpu.VMEM((CHUNK, OUT_DIM), jnp.float32),
            pltpu.SemaphoreType.DMA,
            pltpu.SemaphoreType.DMA,
        ],
        compiler_params=pltpu.CompilerParams(
            dimension_semantics=("arbitrary",),
        ),
    )(x, A, Bm)
    return (out, inner)


# speedup vs baseline: 1.4251x; 1.4251x over previous
import jax
import jax.numpy as jnp
from jax.experimental import pallas as pl
from jax.experimental.pallas import tpu as pltpu

IN_DIM = 1024
OUT_DIM = 1024
K = 2048
B_TOK = 8192
TM = 512


def _fused_body(x_ref, A_ref, B_ref, out_ref, inner_ref, An_s, W_s):
    i = pl.program_id(0)

    @pl.when(i == 0)
    def _prep():
        a = A_ref[...]
        inv = jax.lax.rsqrt(jnp.sum(a * a, axis=0, keepdims=True))
        An_s[...] = (a * inv).astype(jnp.bfloat16)
        W_s[...] = jnp.dot(
            An_s[...], B_ref[...].astype(jnp.bfloat16),
            preferred_element_type=jnp.float32).astype(jnp.bfloat16)

    xb = x_ref[...].astype(jnp.bfloat16)
    inner_ref[...] = jnp.dot(xb, An_s[...], preferred_element_type=jnp.float32)
    out_ref[...] = jnp.dot(xb, W_s[...], preferred_element_type=jnp.float32)


def kernel(x, A, Bm):
    n_tiles = B_TOK // TM
    out, inner = pl.pallas_call(
        _fused_body,
        grid=(n_tiles,),
        in_specs=[
            pl.BlockSpec((TM, IN_DIM), lambda i: (i, 0)),
            pl.BlockSpec((IN_DIM, K), lambda i: (0, 0)),
            pl.BlockSpec((K, OUT_DIM), lambda i: (0, 0)),
        ],
        out_specs=[
            pl.BlockSpec((TM, OUT_DIM), lambda i: (i, 0)),
            pl.BlockSpec((TM, K), lambda i: (i, 0)),
        ],
        out_shape=[
            jax.ShapeDtypeStruct((B_TOK, OUT_DIM), jnp.float32),
            jax.ShapeDtypeStruct((B_TOK, K), jnp.float32),
        ],
        scratch_shapes=[
            pltpu.VMEM((IN_DIM, K), jnp.bfloat16),
            pltpu.VMEM((IN_DIM, OUT_DIM), jnp.bfloat16),
        ],
        compiler_params=pltpu.CompilerParams(
            dimension_semantics=("arbitrary",),
        ),
    )(x, A, Bm)
    return (out, inner)
